# baseline (device time: 47921 ns/iter reference)
import jax
import jax.numpy as jnp
from jax import lax
from jax.experimental import pallas as pl
from jax.experimental.pallas import tpu as pltpu

N_DEV = 4
X_DT = jnp.float8_e4m3fn
W_DT = jnp.float8_e5m2


def _cast(a, dt):
    m, k = a.shape
    T = 4
    tm = m // T

    def body(a_ref, o_ref, in_buf, out_buf, in_sems, out_sems):
        def in_cp(t):
            return pltpu.make_async_copy(
                a_ref.at[pl.ds(t * tm, tm), :], in_buf.at[t % 2],
                in_sems.at[t % 2],
            )

        def out_cp(t):
            return pltpu.make_async_copy(
                out_buf.at[t % 2], o_ref.at[pl.ds(t * tm, tm), :],
                out_sems.at[t % 2],
            )

        in_cp(0).start()
        for t in range(T):
            if t + 1 < T:
                in_cp(t + 1).start()
            in_cp(t).wait()
            if t >= 2:
                out_cp(t - 2).wait()
            out_buf[t % 2] = in_buf[t % 2].astype(dt)
            out_cp(t).start()
        out_cp(T - 2).wait()
        out_cp(T - 1).wait()

    return pl.pallas_call(
        body,
        in_specs=[pl.BlockSpec(memory_space=pl.ANY)],
        out_specs=pl.BlockSpec(memory_space=pl.ANY),
        out_shape=jax.ShapeDtypeStruct((m, k), dt),
        scratch_shapes=[
            pltpu.VMEM((2, tm, k), a.dtype),
            pltpu.VMEM((2, tm, k), dt),
            pltpu.SemaphoreType.DMA((2,)),
            pltpu.SemaphoreType.DMA((2,)),
        ],
    )(a)


def kernel(x, w_mat, scale_x, scale_w):
    m_per, k = x.shape
    _, n = w_mat.shape
    n_per = n // N_DEV

    xq = _cast(x, X_DT)
    my_arr = jnp.full((1,), lax.axis_index("i"), jnp.int32)

    def body(my_ref, x_ref, w_ref, sx_ref, sw_ref, out_ref,
             sendbuf, recvbuf, ssend, srecv, stage,
             send_sems, recv_sems, ssend_sems, srecv_sems, copy_sems):
        j = pl.program_id(0)
        del my_ref
        my = lax.axis_index("i")
        tgt = lax.rem(my + 1 + j, N_DEV)
        s = sx_ref[0] * sw_ref[0]

        wq = w_ref[...].astype(W_DT)
        blk = jnp.maximum(
            jnp.dot(x_ref[...], wq, preferred_element_type=jnp.float32) * s,
            0.0,
        )

        @pl.when(j == N_DEV - 1)
        def _():
            stage[N_DEV - 1] = blk
            pltpu.make_async_copy(
                stage.at[N_DEV - 1],
                out_ref.at[pl.ds(my * m_per, m_per), :],
                copy_sems.at[N_DEV - 1],
            ).start()

        @pl.when(j < N_DEV - 1)
        def _():
            cmax = jnp.max(blk, axis=0, keepdims=True)
            ssend[j] = jnp.maximum(cmax, 1e-30) * (1.0 / 127.0)
            sendbuf[j] = jnp.rint(blk * (127.0 / jnp.maximum(cmax, 1e-30))
                                  ).astype(jnp.int8)
            rdma = pltpu.make_async_remote_copy(
                src_ref=sendbuf.at[j],
                dst_ref=recvbuf.at[my],
                send_sem=send_sems.at[j],
                recv_sem=recv_sems.at[my],
                device_id=(tgt,),
                device_id_type=pl.DeviceIdType.MESH,
            )
            rdma.start()
            srdma = pltpu.make_async_remote_copy(
                src_ref=ssend.at[j],
                dst_ref=srecv.at[my],
                send_sem=ssend_sems.at[j],
                recv_sem=srecv_sems.at[my],
                device_id=(tgt,),
                device_id_type=pl.DeviceIdType.MESH,
            )
            srdma.start()

        def wait_and_store(d):
            src = lax.rem(my + N_DEV - d, N_DEV)
            recv = pltpu.make_async_remote_copy(
                src_ref=sendbuf.at[0],
                dst_ref=recvbuf.at[src],
                send_sem=send_sems.at[0],
                recv_sem=recv_sems.at[src],
                device_id=(0,),
                device_id_type=pl.DeviceIdType.MESH,
            )
            recv.wait_recv()
            srecv_d = pltpu.make_async_remote_copy(
                src_ref=ssend.at[0],
                dst_ref=srecv.at[src],
                send_sem=ssend_sems.at[0],
                recv_sem=srecv_sems.at[src],
                device_id=(0,),
                device_id_type=pl.DeviceIdType.MESH,
            )
            srecv_d.wait_recv()
            stage[d - 1] = recvbuf[src].astype(jnp.float32) * srecv[src]
            pltpu.make_async_copy(
                stage.at[d - 1],
                out_ref.at[pl.ds(src * m_per, m_per), :],
                copy_sems.at[d - 1],
            ).start()

        @pl.when(j == N_DEV - 2)
        def _():
            wait_and_store(1)

        @pl.when(j == N_DEV - 1)
        def _():
            wait_and_store(2)
            wait_and_store(3)
            for slot in range(N_DEV - 1):
                snd = pltpu.make_async_remote_copy(
                    src_ref=sendbuf.at[slot],
                    dst_ref=recvbuf.at[my],
                    send_sem=send_sems.at[slot],
                    recv_sem=recv_sems.at[my],
                    device_id=(0,),
                    device_id_type=pl.DeviceIdType.MESH,
                )
                snd.wait_send()
                ssnd = pltpu.make_async_remote_copy(
                    src_ref=ssend.at[slot],
                    dst_ref=srecv.at[my],
                    send_sem=ssend_sems.at[slot],
                    recv_sem=srecv_sems.at[my],
                    device_id=(0,),
                    device_id_type=pl.DeviceIdType.MESH,
                )
                ssnd.wait_send()

            for slot in range(N_DEV):
                rows = (
                    my if slot == N_DEV - 1
                    else lax.rem(my + N_DEV - 1 - slot, N_DEV)
                )
                pltpu.make_async_copy(
                    stage.at[slot],
                    out_ref.at[pl.ds(rows * m_per, m_per), :],
                    copy_sems.at[slot],
                ).wait()

    grid_spec = pltpu.PrefetchScalarGridSpec(
        num_scalar_prefetch=1,
        grid=(N_DEV,),
        in_specs=[
            pl.BlockSpec((m_per, k), lambda j, my: (0, 0)),
            pl.BlockSpec(
                (k, n_per), lambda j, my: (0, lax.rem(my[0] + 1 + j, N_DEV))
            ),
            pl.BlockSpec(memory_space=pltpu.SMEM),
            pl.BlockSpec(memory_space=pltpu.SMEM),
        ],
        out_specs=pl.BlockSpec(memory_space=pl.ANY),
        scratch_shapes=[
            pltpu.VMEM((N_DEV - 1, m_per, n_per), jnp.int8),
            pltpu.VMEM((N_DEV, m_per, n_per), jnp.int8),
            pltpu.VMEM((N_DEV - 1, 1, n_per), jnp.float32),
            pltpu.VMEM((N_DEV, 1, n_per), jnp.float32),
            pltpu.VMEM((N_DEV, m_per, n_per), jnp.float32),
            pltpu.SemaphoreType.DMA((N_DEV - 1,)),
            pltpu.SemaphoreType.DMA((N_DEV,)),
            pltpu.SemaphoreType.DMA((N_DEV - 1,)),
            pltpu.SemaphoreType.DMA((N_DEV,)),
            pltpu.SemaphoreType.DMA((N_DEV,)),
        ],
    )

    out_shape = jax.ShapeDtypeStruct((N_DEV * m_per, n_per), jnp.float32)
    return pl.pallas_call(
        body,
        grid_spec=grid_spec,
        out_shape=out_shape,
        compiler_params=pltpu.CompilerParams(
            dimension_semantics=("arbitrary",),
            vmem_limit_bytes=63 * 1024 * 1024,
        ),
    )(my_arr, xq, w_mat, scale_x, scale_w)


# device time: 46483 ns/iter; 1.0309x vs baseline; 1.0309x over previous
import jax
import jax.numpy as jnp
from jax import lax
from jax.experimental import pallas as pl
from jax.experimental.pallas import tpu as pltpu

N_DEV = 4
X_DT = jnp.float8_e4m3fn
W_DT = jnp.float8_e5m2


def _cast(a, dt):
    m, k = a.shape

    def body(a_ref, o_ref):
        o_ref[...] = a_ref[...].astype(dt)

    return pl.pallas_call(
        body,
        grid=(4,),
        in_specs=[pl.BlockSpec((m // 4, k), lambda i: (i, 0))],
        out_specs=pl.BlockSpec((m // 4, k), lambda i: (i, 0)),
        out_shape=jax.ShapeDtypeStruct((m, k), dt),
    )(a)


def kernel(x, w_mat, scale_x, scale_w):
    m_per, k = x.shape
    _, n = w_mat.shape
    n_per = n // N_DEV

    xq = _cast(x, X_DT)
    my_arr = jnp.full((1,), lax.axis_index("i"), jnp.int32)

    def body(my_ref, x_ref, w_ref, sx_ref, sw_ref, out_ref,
             sendbuf, recvbuf, ssend, srecv, stage,
             send_sems, recv_sems, ssend_sems, srecv_sems, copy_sems):
        j = pl.program_id(0)
        del my_ref
        my = lax.axis_index("i")
        tgt = lax.rem(my + 1 + j, N_DEV)
        s = sx_ref[0] * sw_ref[0]

        wq = w_ref[...].astype(W_DT)
        blk = jnp.maximum(
            jnp.dot(x_ref[...], wq, preferred_element_type=jnp.float32) * s,
            0.0,
        )

        @pl.when(j == N_DEV - 1)
        def _():
            stage[N_DEV - 1] = blk
            pltpu.make_async_copy(
                stage.at[N_DEV - 1],
                out_ref.at[pl.ds(my * m_per, m_per), :],
                copy_sems.at[N_DEV - 1],
            ).start()

        @pl.when(j < N_DEV - 1)
        def _():
            cmax = jnp.max(blk, axis=0, keepdims=True)
            ssend[j] = jnp.maximum(cmax, 1e-30) * (1.0 / 127.0)
            sendbuf[j] = jnp.rint(blk * (127.0 / jnp.maximum(cmax, 1e-30))
                                  ).astype(jnp.int8)
            rdma = pltpu.make_async_remote_copy(
                src_ref=sendbuf.at[j],
                dst_ref=recvbuf.at[my],
                send_sem=send_sems.at[j],
                recv_sem=recv_sems.at[my],
                device_id=(tgt,),
                device_id_type=pl.DeviceIdType.MESH,
            )
            rdma.start()
            srdma = pltpu.make_async_remote_copy(
                src_ref=ssend.at[j],
                dst_ref=srecv.at[my],
                send_sem=ssend_sems.at[j],
                recv_sem=srecv_sems.at[my],
                device_id=(tgt,),
                device_id_type=pl.DeviceIdType.MESH,
            )
            srdma.start()

        def wait_and_store(d):
            src = lax.rem(my + N_DEV - d, N_DEV)
            recv = pltpu.make_async_remote_copy(
                src_ref=sendbuf.at[0],
                dst_ref=recvbuf.at[src],
                send_sem=send_sems.at[0],
                recv_sem=recv_sems.at[src],
                device_id=(0,),
                device_id_type=pl.DeviceIdType.MESH,
            )
            recv.wait_recv()
            srecv_d = pltpu.make_async_remote_copy(
                src_ref=ssend.at[0],
                dst_ref=srecv.at[src],
                send_sem=ssend_sems.at[0],
                recv_sem=srecv_sems.at[src],
                device_id=(0,),
                device_id_type=pl.DeviceIdType.MESH,
            )
            srecv_d.wait_recv()
            stage[d - 1] = recvbuf[src].astype(jnp.float32) * srecv[src]
            pltpu.make_async_copy(
                stage.at[d - 1],
                out_ref.at[pl.ds(src * m_per, m_per), :],
                copy_sems.at[d - 1],
            ).start()

        @pl.when(j == N_DEV - 2)
        def _():
            wait_and_store(1)

        @pl.when(j == N_DEV - 1)
        def _():
            wait_and_store(2)
            wait_and_store(3)
            for slot in range(N_DEV - 1):
                snd = pltpu.make_async_remote_copy(
                    src_ref=sendbuf.at[slot],
                    dst_ref=recvbuf.at[my],
                    send_sem=send_sems.at[slot],
                    recv_sem=recv_sems.at[my],
                    device_id=(0,),
                    device_id_type=pl.DeviceIdType.MESH,
                )
                snd.wait_send()
                ssnd = pltpu.make_async_remote_copy(
                    src_ref=ssend.at[slot],
                    dst_ref=srecv.at[my],
                    send_sem=ssend_sems.at[slot],
                    recv_sem=srecv_sems.at[my],
                    device_id=(0,),
                    device_id_type=pl.DeviceIdType.MESH,
                )
                ssnd.wait_send()

            for slot in range(N_DEV):
                rows = (
                    my if slot == N_DEV - 1
                    else lax.rem(my + N_DEV - 1 - slot, N_DEV)
                )
                pltpu.make_async_copy(
                    stage.at[slot],
                    out_ref.at[pl.ds(rows * m_per, m_per), :],
                    copy_sems.at[slot],
                ).wait()

    grid_spec = pltpu.PrefetchScalarGridSpec(
        num_scalar_prefetch=1,
        grid=(N_DEV,),
        in_specs=[
            pl.BlockSpec((m_per, k), lambda j, my: (0, 0)),
            pl.BlockSpec(
                (k, n_per), lambda j, my: (0, lax.rem(my[0] + 1 + j, N_DEV))
            ),
            pl.BlockSpec(memory_space=pltpu.SMEM),
            pl.BlockSpec(memory_space=pltpu.SMEM),
        ],
        out_specs=pl.BlockSpec(memory_space=pl.ANY),
        scratch_shapes=[
            pltpu.VMEM((N_DEV - 1, m_per, n_per), jnp.int8),
            pltpu.VMEM((N_DEV, m_per, n_per), jnp.int8),
            pltpu.VMEM((N_DEV - 1, 1, n_per), jnp.float32),
            pltpu.VMEM((N_DEV, 1, n_per), jnp.float32),
            pltpu.VMEM((N_DEV, m_per, n_per), jnp.float32),
            pltpu.SemaphoreType.DMA((N_DEV - 1,)),
            pltpu.SemaphoreType.DMA((N_DEV,)),
            pltpu.SemaphoreType.DMA((N_DEV - 1,)),
            pltpu.SemaphoreType.DMA((N_DEV,)),
            pltpu.SemaphoreType.DMA((N_DEV,)),
        ],
    )

    out_shape = jax.ShapeDtypeStruct((N_DEV * m_per, n_per), jnp.float32)
    return pl.pallas_call(
        body,
        grid_spec=grid_spec,
        out_shape=out_shape,
        compiler_params=pltpu.CompilerParams(
            dimension_semantics=("arbitrary",),
            vmem_limit_bytes=63 * 1024 * 1024,
        ),
    )(my_arr, xq, w_mat, scale_x, scale_w)
